# baseline (speedup 1.0000x reference)
import functools

import jax
import jax.numpy as jnp
from jax import lax
from jax.experimental import pallas as pl
from jax.experimental.pallas import tpu as pltpu
from jax.experimental.pallas import tpu_sc as plsc

_N0, _N1, _N2 = 50000, 10000, 1000
_E0, _E1 = 160000, 16000
_D_IN, _D_HID, _D_OUT = 256, 128, 64

_NC, _NS = 2, 16
_BATCH = 128
_CW = 16
_R = 4


def _make_seg_sum(e_pad, dh, s_pad):
    nb = e_pad // (_NS * _BATCH)
    zr = s_pad // _NS
    zc = min(_BATCH, zr)
    mesh = plsc.VectorSubcoreMesh(core_axis_name="c", subcore_axis_name="s")

    @functools.partial(
        pl.kernel,
        mesh=mesh,
        compiler_params=pltpu.CompilerParams(use_tc_tiling_on_sc=False),
        out_type=[
            jax.ShapeDtypeStruct((_NC, s_pad, dh), jnp.float32),
            jax.ShapeDtypeStruct((_NC, s_pad, _CW), jnp.float32),
        ],
        scratch_types=[
            pltpu.VMEM((nb, _BATCH), jnp.int32),
            pltpu.VMEM((nb, _BATCH), jnp.int32),
            pltpu.VMEM((_R, _BATCH, dh), jnp.float32),
            pltpu.VMEM((_BATCH, _CW), jnp.float32),
            pltpu.VMEM((_BATCH, _CW), jnp.float32),
            [pltpu.SemaphoreType.DMA] * _R,
            [pltpu.SemaphoreType.DMA] * _R,
            pltpu.SemaphoreType.DMA,
            pltpu.VMEM_SHARED((s_pad, dh), jnp.float32),
            pltpu.VMEM_SHARED((s_pad, _CW), jnp.float32),
        ],
    )
    def seg_kernel(table, src, dst, zeros_d, zeros_c, ones_c, acc_out, cnt_out,
                   src_v, dst_v, rows_v, ones_v, zc_v, gsems, ssems, csem,
                   acc_sh, cnt_sh):
        c = lax.axis_index("c")
        s = lax.axis_index("s")
        pltpu.sync_copy(zeros_d, rows_v.at[0])
        pltpu.sync_copy(zeros_c, zc_v)
        pltpu.sync_copy(ones_c, ones_v)
        for t in range(zr // zc):
            r0 = s * zr + t * zc
            pltpu.sync_copy(rows_v.at[0, pl.ds(0, zc)], acc_sh.at[pl.ds(r0, zc)])
            pltpu.sync_copy(zc_v.at[pl.ds(0, zc)], cnt_sh.at[pl.ds(r0, zc)])
        plsc.subcore_barrier()
        base = s * nb
        pltpu.sync_copy(src.at[c, pl.ds(base, nb)], src_v)
        pltpu.sync_copy(dst.at[c, pl.ds(base, nb)], dst_v)
        tbl = table
        pltpu.async_copy(tbl.at[src_v.at[0]], rows_v.at[0], gsems[0])
        pltpu.async_copy(tbl.at[src_v.at[1]], rows_v.at[1], gsems[1])

        def body(g, carry):
            for r in range(_R):
                j = _R * g + r
                jj = j + 2
                rr = (r + 2) % _R

                @pl.when(jj < nb)
                def _():
                    @pl.when(jj >= _R)
                    def _():
                        pltpu.make_async_copy(
                            rows_v.at[rr], acc_sh.at[dst_v.at[jj - _R]],
                            ssems[rr]).wait()
                    pltpu.async_copy(
                        tbl.at[src_v.at[jj]], rows_v.at[rr], gsems[rr])

                pltpu.make_async_copy(
                    tbl.at[src_v.at[j]], rows_v.at[r], gsems[r]).wait()
                pltpu.async_copy(rows_v.at[r], acc_sh.at[dst_v.at[j]],
                                 ssems[r], add=True)
                pltpu.async_copy(ones_v, cnt_sh.at[dst_v.at[j]], csem,
                                 add=True)
            return carry

        lax.fori_loop(0, nb // _R, body, 0)
        for r in range(_R):
            pltpu.make_async_copy(
                rows_v.at[r], acc_sh.at[dst_v.at[nb - _R + r]],
                ssems[r]).wait()
        for _ in range(nb):
            pltpu.make_async_copy(ones_v, cnt_sh.at[dst_v.at[0]], csem).wait()
        plsc.subcore_barrier()
        r0 = s * zr
        pltpu.sync_copy(acc_sh.at[pl.ds(r0, zr)], acc_out.at[c, pl.ds(r0, zr)])
        pltpu.sync_copy(cnt_sh.at[pl.ds(r0, zr)], cnt_out.at[c, pl.ds(r0, zr)])

    return seg_kernel


def _matmul(x, w):
    n, k = x.shape
    bm = 2000

    def mk(x_ref, w_ref, o_ref):
        o_ref[...] = jnp.dot(x_ref[...], w_ref[...],
                             preferred_element_type=jnp.float32)

    return pl.pallas_call(
        mk,
        grid=(n // bm,),
        in_specs=[pl.BlockSpec((bm, k), lambda i: (i, 0)),
                  pl.BlockSpec((k, _D_HID), lambda i: (0, 0))],
        out_specs=pl.BlockSpec((bm, _D_HID), lambda i: (i, 0)),
        out_shape=jax.ShapeDtypeStruct((n, _D_HID), jnp.float32),
    )(x, w)


def _mid(acc, cnt, b0, w1a, w1b):
    s_pad = acc.shape[1]
    bm = 1024
    h = _D_OUT // 2

    def mk(a_ref, c_ref, b0r, wa, wb, o):
        inv = 2.0 / jnp.maximum(c_ref[0, :, 0:1] + c_ref[1, :, 0:1], 2.0)
        a = (jnp.concatenate([a_ref[0], a_ref[1]], axis=1) * inv + b0r[...])
        res = (jnp.dot(a, wa[...], preferred_element_type=jnp.float32)
               + jnp.dot(jnp.maximum(a, 0.0), wb[...],
                         preferred_element_type=jnp.float32))
        o[...] = jnp.concatenate(
            [res, jnp.zeros((bm, _D_HID - _D_OUT), jnp.float32)], axis=1)

    return pl.pallas_call(
        mk,
        grid=(s_pad // bm,),
        in_specs=[pl.BlockSpec((2, bm, _D_HID // 2), lambda i: (0, i, 0)),
                  pl.BlockSpec((2, bm, _CW), lambda i: (0, i, 0)),
                  pl.BlockSpec((1, _D_HID), lambda i: (0, 0)),
                  pl.BlockSpec((_D_HID, _D_OUT), lambda i: (0, 0)),
                  pl.BlockSpec((_D_HID, _D_OUT), lambda i: (0, 0))],
        out_specs=pl.BlockSpec((bm, _D_HID), lambda i: (i, 0)),
        out_shape=jax.ShapeDtypeStruct((s_pad, _D_HID), jnp.float32),
    )(acc, cnt, b0, w1a, w1b)


def _fin(acc, cnt, b1):
    s_pad = acc.shape[1]

    def mk(a_ref, c_ref, b1r, o):
        inv = 2.0 / jnp.maximum(c_ref[0, :, 0:1] + c_ref[1, :, 0:1], 2.0)
        res = (jnp.concatenate([a_ref[0], a_ref[1]], axis=1) * inv + b1r[...])
        o[...] = res[:_N2]

    return pl.pallas_call(
        mk,
        grid=(1,),
        in_specs=[pl.BlockSpec((2, s_pad, _D_OUT // 2), lambda i: (0, 0, 0)),
                  pl.BlockSpec((2, s_pad, _CW), lambda i: (0, 0, 0)),
                  pl.BlockSpec((1, _D_OUT), lambda i: (0, 0))],
        out_specs=pl.BlockSpec((_N2, _D_OUT), lambda i: (0, 0)),
        out_shape=jax.ShapeDtypeStruct((_N2, _D_OUT), jnp.float32),
    )(acc, cnt, b1)


_E0_PAD = 163840
_E1_PAD = 16384
_S0_PAD = 10240
_S1_PAD = 1024


@functools.lru_cache(maxsize=None)
def _seg_sum(e_pad, dh, s_pad):
    return _make_seg_sum(e_pad, dh, s_pad)


def _pad_edges(src, dst, e, e_pad, dummy_dst, stride):
    nbt = e_pad // _BATCH
    roll = e_pad // (2 * _NS)
    srcp = jnp.concatenate([src, jnp.zeros((e_pad - e,), jnp.int32)])
    dstp = jnp.concatenate([dst, jnp.full((e_pad - e,), dummy_dst, jnp.int32)])
    srcq = jnp.stack([stride * srcp,
                      jnp.roll(stride * srcp + 1, roll)]).reshape(
        2, nbt, _BATCH)
    dstq = jnp.stack([dstp, jnp.roll(dstp, roll)]).reshape(2, nbt, _BATCH)
    return srcq, dstq


def kernel(features, src0, dst0, src1, dst1, W0, b0, W1, b1):
    src0q, dst0p = _pad_edges(src0, dst0, _E0, _E0_PAD, _N1, 2)
    src1q, dst1p = _pad_edges(src1, dst1, _E1, _E1_PAD, _N2, 4)
    zeros_h = jnp.zeros((_BATCH, _D_HID // 2), jnp.float32)
    zeros_o = jnp.zeros((_BATCH, _D_OUT // 2), jnp.float32)
    zeros_c = jnp.zeros((_BATCH, _CW), jnp.float32)
    ones_c = jnp.ones((_BATCH, _CW), jnp.float32)

    f = _matmul(features, W0)
    tbl0 = f.reshape(2 * _N0, _D_HID // 2)
    acc0, cnt0 = _seg_sum(_E0_PAD, _D_HID // 2, _S0_PAD)(
        tbl0, src0q, dst0p, zeros_h, zeros_c, ones_c)
    g = _mid(acc0, cnt0, b0.reshape(1, _D_HID),
             W1[:_D_HID], W1[_D_HID:])
    tbl1 = g.reshape(4 * _S0_PAD, _D_OUT // 2)
    acc1, cnt1 = _seg_sum(_E1_PAD, _D_OUT // 2, _S1_PAD)(
        tbl1, src1q, dst1p, zeros_o, zeros_c, ones_c)
    return _fin(acc1, cnt1, b1.reshape(1, _D_OUT))


# speedup vs baseline: 1.0005x; 1.0005x over previous
import functools

import jax
import jax.numpy as jnp
from jax import lax
from jax.experimental import pallas as pl
from jax.experimental.pallas import tpu as pltpu
from jax.experimental.pallas import tpu_sc as plsc

_N0, _N1, _N2 = 50000, 10000, 1000
_E0, _E1 = 160000, 16000
_D_IN, _D_HID, _D_OUT = 256, 128, 64

_NC, _NS = 2, 16
_BATCH = 128
_CW = 16
_R = 4


def _make_seg_sum(e_pad, dh, s_pad):
    nb = e_pad // (_NS * _BATCH)
    zr = s_pad // _NS
    zc = min(_BATCH, zr)
    mesh = plsc.VectorSubcoreMesh(core_axis_name="c", subcore_axis_name="s")

    @functools.partial(
        pl.kernel,
        mesh=mesh,
        compiler_params=pltpu.CompilerParams(use_tc_tiling_on_sc=False),
        out_type=[
            jax.ShapeDtypeStruct((_NC, s_pad, dh), jnp.float32),
            jax.ShapeDtypeStruct((_NC, s_pad, _CW), jnp.float32),
        ],
        scratch_types=[
            pltpu.VMEM((nb, _BATCH), jnp.int32),
            pltpu.VMEM((nb, _BATCH), jnp.int32),
            pltpu.VMEM((_R, _BATCH, dh), jnp.float32),
            pltpu.VMEM((_BATCH, _CW), jnp.float32),
            pltpu.VMEM((_BATCH, _CW), jnp.float32),
            [pltpu.SemaphoreType.DMA] * _R,
            [pltpu.SemaphoreType.DMA] * _R,
            pltpu.SemaphoreType.DMA,
            pltpu.VMEM_SHARED((s_pad, dh), jnp.float32),
            pltpu.VMEM_SHARED((s_pad, _CW), jnp.float32),
        ],
    )
    def seg_kernel(table, src, dst, zeros_d, zeros_c, ones_c, acc_out, cnt_out,
                   src_v, dst_v, rows_v, ones_v, zc_v, gsems, ssems, csem,
                   acc_sh, cnt_sh):
        c = lax.axis_index("c")
        s = lax.axis_index("s")
        pltpu.sync_copy(zeros_d, rows_v.at[0])
        pltpu.sync_copy(zeros_c, zc_v)
        pltpu.sync_copy(ones_c, ones_v)
        for t in range(zr // zc):
            r0 = s * zr + t * zc
            pltpu.sync_copy(rows_v.at[0, pl.ds(0, zc)], acc_sh.at[pl.ds(r0, zc)])
            pltpu.sync_copy(zc_v.at[pl.ds(0, zc)], cnt_sh.at[pl.ds(r0, zc)])
        plsc.subcore_barrier()
        base = s * nb
        pltpu.sync_copy(src.at[c, pl.ds(base, nb)], src_v)
        pltpu.sync_copy(dst.at[c, pl.ds(base, nb)], dst_v)
        tbl = table
        pltpu.async_copy(tbl.at[src_v.at[0]], rows_v.at[0], gsems[0])
        pltpu.async_copy(tbl.at[src_v.at[1]], rows_v.at[1], gsems[1])

        def body(g, carry):
            for r in range(_R):
                j = _R * g + r
                jj = j + 2
                rr = (r + 2) % _R

                @pl.when(jj < nb)
                def _():
                    @pl.when(jj >= _R)
                    def _():
                        pltpu.make_async_copy(
                            rows_v.at[rr], acc_sh.at[dst_v.at[jj - _R]],
                            ssems[rr]).wait()
                    pltpu.async_copy(
                        tbl.at[src_v.at[jj]], rows_v.at[rr], gsems[rr])

                pltpu.make_async_copy(
                    tbl.at[src_v.at[j]], rows_v.at[r], gsems[r]).wait()
                pltpu.async_copy(rows_v.at[r], acc_sh.at[dst_v.at[j]],
                                 ssems[r], add=True)

                @pl.when(j < nb // 2)
                def _():
                    pltpu.async_copy(ones_v, cnt_sh.at[dst_v.at[j]], csem,
                                     add=True)
            return carry

        lax.fori_loop(0, nb // _R, body, 0)
        for r in range(_R):
            pltpu.make_async_copy(
                rows_v.at[r], acc_sh.at[dst_v.at[nb - _R + r]],
                ssems[r]).wait()
        for _ in range(nb // 2):
            pltpu.make_async_copy(ones_v, cnt_sh.at[dst_v.at[0]], csem).wait()
        plsc.subcore_barrier()
        r0 = s * zr
        pltpu.sync_copy(acc_sh.at[pl.ds(r0, zr)], acc_out.at[c, pl.ds(r0, zr)])
        pltpu.sync_copy(cnt_sh.at[pl.ds(r0, zr)], cnt_out.at[c, pl.ds(r0, zr)])

    return seg_kernel


def _matmul(x, w):
    n, k = x.shape
    bm = 2000

    def mk(x_ref, w_ref, o_ref):
        o_ref[...] = jnp.dot(x_ref[...], w_ref[...],
                             preferred_element_type=jnp.float32)

    return pl.pallas_call(
        mk,
        grid=(n // bm,),
        in_specs=[pl.BlockSpec((bm, k), lambda i: (i, 0)),
                  pl.BlockSpec((k, _D_HID), lambda i: (0, 0))],
        out_specs=pl.BlockSpec((bm, _D_HID), lambda i: (i, 0)),
        out_shape=jax.ShapeDtypeStruct((n, _D_HID), jnp.float32),
    )(x, w)


def _mid(acc, cnt, b0, w1a, w1b):
    s_pad = acc.shape[1]
    bm = 1024
    h = _D_OUT // 2

    def mk(a_ref, c_ref, b0r, wa, wb, o):
        inv = 1.0 / jnp.maximum(c_ref[0, :, 0:1] + c_ref[1, :, 0:1], 1.0)
        a = (jnp.concatenate([a_ref[0], a_ref[1]], axis=1) * inv + b0r[...])
        res = (jnp.dot(a, wa[...], preferred_element_type=jnp.float32)
               + jnp.dot(jnp.maximum(a, 0.0), wb[...],
                         preferred_element_type=jnp.float32))
        o[...] = jnp.concatenate(
            [res, jnp.zeros((bm, _D_HID - _D_OUT), jnp.float32)], axis=1)

    return pl.pallas_call(
        mk,
        grid=(s_pad // bm,),
        in_specs=[pl.BlockSpec((2, bm, _D_HID // 2), lambda i: (0, i, 0)),
                  pl.BlockSpec((2, bm, _CW), lambda i: (0, i, 0)),
                  pl.BlockSpec((1, _D_HID), lambda i: (0, 0)),
                  pl.BlockSpec((_D_HID, _D_OUT), lambda i: (0, 0)),
                  pl.BlockSpec((_D_HID, _D_OUT), lambda i: (0, 0))],
        out_specs=pl.BlockSpec((bm, _D_HID), lambda i: (i, 0)),
        out_shape=jax.ShapeDtypeStruct((s_pad, _D_HID), jnp.float32),
    )(acc, cnt, b0, w1a, w1b)


def _fin(acc, cnt, b1):
    s_pad = acc.shape[1]

    def mk(a_ref, c_ref, b1r, o):
        inv = 1.0 / jnp.maximum(c_ref[0, :, 0:1] + c_ref[1, :, 0:1], 1.0)
        res = (jnp.concatenate([a_ref[0], a_ref[1]], axis=1) * inv + b1r[...])
        o[...] = res[:_N2]

    return pl.pallas_call(
        mk,
        grid=(1,),
        in_specs=[pl.BlockSpec((2, s_pad, _D_OUT // 2), lambda i: (0, 0, 0)),
                  pl.BlockSpec((2, s_pad, _CW), lambda i: (0, 0, 0)),
                  pl.BlockSpec((1, _D_OUT), lambda i: (0, 0))],
        out_specs=pl.BlockSpec((_N2, _D_OUT), lambda i: (0, 0)),
        out_shape=jax.ShapeDtypeStruct((_N2, _D_OUT), jnp.float32),
    )(acc, cnt, b1)


_E0_PAD = 163840
_E1_PAD = 16384
_S0_PAD = 10240
_S1_PAD = 1024


@functools.lru_cache(maxsize=None)
def _seg_sum(e_pad, dh, s_pad):
    return _make_seg_sum(e_pad, dh, s_pad)


def _pad_edges(src, dst, e, e_pad, dummy_dst, stride):
    nbt = e_pad // _BATCH
    roll = e_pad // (2 * _NS)
    srcp = jnp.concatenate([src, jnp.zeros((e_pad - e,), jnp.int32)])
    dstp = jnp.concatenate([dst, jnp.full((e_pad - e,), dummy_dst, jnp.int32)])
    srcq = jnp.stack([stride * srcp,
                      jnp.roll(stride * srcp + 1, roll)]).reshape(
        2, nbt, _BATCH)
    dstq = jnp.stack([dstp, jnp.roll(dstp, roll)]).reshape(2, nbt, _BATCH)
    return srcq, dstq


def kernel(features, src0, dst0, src1, dst1, W0, b0, W1, b1):
    src0q, dst0p = _pad_edges(src0, dst0, _E0, _E0_PAD, _N1, 2)
    src1q, dst1p = _pad_edges(src1, dst1, _E1, _E1_PAD, _N2, 4)
    zeros_h = jnp.zeros((_BATCH, _D_HID // 2), jnp.float32)
    zeros_o = jnp.zeros((_BATCH, _D_OUT // 2), jnp.float32)
    zeros_c = jnp.zeros((_BATCH, _CW), jnp.float32)
    ones_c = jnp.ones((_BATCH, _CW), jnp.float32)

    f = _matmul(features, W0)
    tbl0 = f.reshape(2 * _N0, _D_HID // 2)
    acc0, cnt0 = _seg_sum(_E0_PAD, _D_HID // 2, _S0_PAD)(
        tbl0, src0q, dst0p, zeros_h, zeros_c, ones_c)
    g = _mid(acc0, cnt0, b0.reshape(1, _D_HID),
             W1[:_D_HID], W1[_D_HID:])
    tbl1 = g.reshape(4 * _S0_PAD, _D_OUT // 2)
    acc1, cnt1 = _seg_sum(_E1_PAD, _D_OUT // 2, _S1_PAD)(
        tbl1, src1q, dst1p, zeros_o, zeros_c, ones_c)
    return _fin(acc1, cnt1, b1.reshape(1, _D_OUT))


# speedup vs baseline: 1.0007x; 1.0002x over previous
import functools

import jax
import jax.numpy as jnp
from jax import lax
from jax.experimental import pallas as pl
from jax.experimental.pallas import tpu as pltpu
from jax.experimental.pallas import tpu_sc as plsc

_N0, _N1, _N2 = 50000, 10000, 1000
_E0, _E1 = 160000, 16000
_D_IN, _D_HID, _D_OUT = 256, 128, 64

_NC, _NS = 2, 16
_BATCH = 128
_CW = 16
_R = 4


def _make_seg_sum(e_pad, dh, s_pad):
    nb = e_pad // (_NS * _BATCH)
    zr = s_pad // _NS
    zc = min(_BATCH, zr)
    mesh = plsc.VectorSubcoreMesh(core_axis_name="c", subcore_axis_name="s")

    @functools.partial(
        pl.kernel,
        mesh=mesh,
        compiler_params=pltpu.CompilerParams(use_tc_tiling_on_sc=False),
        out_type=[
            jax.ShapeDtypeStruct((_NC, s_pad, dh), jnp.float32),
            jax.ShapeDtypeStruct((_NC, s_pad, _CW), jnp.float32),
        ],
        scratch_types=[
            pltpu.VMEM((nb, _BATCH), jnp.int32),
            pltpu.VMEM((nb, _BATCH), jnp.int32),
            pltpu.VMEM((_R, _BATCH, dh), jnp.float32),
            pltpu.VMEM((_BATCH, _CW), jnp.float32),
            pltpu.VMEM((_BATCH, _CW), jnp.float32),
            [pltpu.SemaphoreType.DMA] * _R,
            [pltpu.SemaphoreType.DMA] * _R,
            pltpu.SemaphoreType.DMA,
            pltpu.VMEM_SHARED((s_pad, dh), jnp.float32),
            pltpu.VMEM_SHARED((s_pad, _CW), jnp.float32),
        ],
    )
    def seg_kernel(table, src, dst, zeros_d, zeros_c, ones_c, acc_out, cnt_out,
                   src_v, dst_v, rows_v, ones_v, zc_v, gsems, ssems, csem,
                   acc_sh, cnt_sh):
        c = lax.axis_index("c")
        s = lax.axis_index("s")
        pltpu.sync_copy(zeros_d, rows_v.at[0])
        pltpu.sync_copy(zeros_c, zc_v)
        pltpu.sync_copy(ones_c, ones_v)
        for t in range(zr // zc):
            r0 = s * zr + t * zc
            pltpu.sync_copy(rows_v.at[0, pl.ds(0, zc)], acc_sh.at[pl.ds(r0, zc)])
            pltpu.sync_copy(zc_v.at[pl.ds(0, zc)], cnt_sh.at[pl.ds(r0, zc)])
        plsc.subcore_barrier()
        base = s * nb
        pltpu.sync_copy(src.at[c, pl.ds(base, nb)], src_v)
        pltpu.sync_copy(dst.at[c, pl.ds(base, nb)], dst_v)
        tbl = table
        pltpu.async_copy(tbl.at[src_v.at[0]], rows_v.at[0], gsems[0])
        pltpu.async_copy(tbl.at[src_v.at[1]], rows_v.at[1], gsems[1])
        pltpu.async_copy(tbl.at[src_v.at[2]], rows_v.at[2], gsems[2])

        def body(g, carry):
            for r in range(_R):
                j = _R * g + r
                jj = j + 3
                rr = (r + 3) % _R

                @pl.when(jj < nb)
                def _():
                    @pl.when(jj >= _R)
                    def _():
                        pltpu.make_async_copy(
                            rows_v.at[rr], acc_sh.at[dst_v.at[jj - _R]],
                            ssems[rr]).wait()
                    pltpu.async_copy(
                        tbl.at[src_v.at[jj]], rows_v.at[rr], gsems[rr])

                pltpu.make_async_copy(
                    tbl.at[src_v.at[j]], rows_v.at[r], gsems[r]).wait()
                pltpu.async_copy(rows_v.at[r], acc_sh.at[dst_v.at[j]],
                                 ssems[r], add=True)

                @pl.when(j < nb // 2)
                def _():
                    pltpu.async_copy(ones_v, cnt_sh.at[dst_v.at[j]], csem,
                                     add=True)
            return carry

        lax.fori_loop(0, nb // _R, body, 0)
        for r in range(_R):
            pltpu.make_async_copy(
                rows_v.at[r], acc_sh.at[dst_v.at[nb - _R + r]],
                ssems[r]).wait()
        for _ in range(nb // 2):
            pltpu.make_async_copy(ones_v, cnt_sh.at[dst_v.at[0]], csem).wait()
        plsc.subcore_barrier()
        r0 = s * zr
        pltpu.sync_copy(acc_sh.at[pl.ds(r0, zr)], acc_out.at[c, pl.ds(r0, zr)])
        pltpu.sync_copy(cnt_sh.at[pl.ds(r0, zr)], cnt_out.at[c, pl.ds(r0, zr)])

    return seg_kernel


def _matmul(x, w):
    n, k = x.shape
    bm = 2000

    def mk(x_ref, w_ref, o_ref):
        o_ref[...] = jnp.dot(x_ref[...], w_ref[...],
                             preferred_element_type=jnp.float32)

    return pl.pallas_call(
        mk,
        grid=(n // bm,),
        in_specs=[pl.BlockSpec((bm, k), lambda i: (i, 0)),
                  pl.BlockSpec((k, _D_HID), lambda i: (0, 0))],
        out_specs=pl.BlockSpec((bm, _D_HID), lambda i: (i, 0)),
        out_shape=jax.ShapeDtypeStruct((n, _D_HID), jnp.float32),
    )(x, w)


def _mid(acc, cnt, b0, w1a, w1b):
    s_pad = acc.shape[1]
    bm = 1024
    h = _D_OUT // 2

    def mk(a_ref, c_ref, b0r, wa, wb, o):
        inv = 1.0 / jnp.maximum(c_ref[0, :, 0:1] + c_ref[1, :, 0:1], 1.0)
        a = (jnp.concatenate([a_ref[0], a_ref[1]], axis=1) * inv + b0r[...])
        res = (jnp.dot(a, wa[...], preferred_element_type=jnp.float32)
               + jnp.dot(jnp.maximum(a, 0.0), wb[...],
                         preferred_element_type=jnp.float32))
        o[...] = jnp.concatenate(
            [res, jnp.zeros((bm, _D_HID - _D_OUT), jnp.float32)], axis=1)

    return pl.pallas_call(
        mk,
        grid=(s_pad // bm,),
        in_specs=[pl.BlockSpec((2, bm, _D_HID // 2), lambda i: (0, i, 0)),
                  pl.BlockSpec((2, bm, _CW), lambda i: (0, i, 0)),
                  pl.BlockSpec((1, _D_HID), lambda i: (0, 0)),
                  pl.BlockSpec((_D_HID, _D_OUT), lambda i: (0, 0)),
                  pl.BlockSpec((_D_HID, _D_OUT), lambda i: (0, 0))],
        out_specs=pl.BlockSpec((bm, _D_HID), lambda i: (i, 0)),
        out_shape=jax.ShapeDtypeStruct((s_pad, _D_HID), jnp.float32),
    )(acc, cnt, b0, w1a, w1b)


def _fin(acc, cnt, b1):
    s_pad = acc.shape[1]

    def mk(a_ref, c_ref, b1r, o):
        inv = 1.0 / jnp.maximum(c_ref[0, :, 0:1] + c_ref[1, :, 0:1], 1.0)
        res = (jnp.concatenate([a_ref[0], a_ref[1]], axis=1) * inv + b1r[...])
        o[...] = res[:_N2]

    return pl.pallas_call(
        mk,
        grid=(1,),
        in_specs=[pl.BlockSpec((2, s_pad, _D_OUT // 2), lambda i: (0, 0, 0)),
                  pl.BlockSpec((2, s_pad, _CW), lambda i: (0, 0, 0)),
                  pl.BlockSpec((1, _D_OUT), lambda i: (0, 0))],
        out_specs=pl.BlockSpec((_N2, _D_OUT), lambda i: (0, 0)),
        out_shape=jax.ShapeDtypeStruct((_N2, _D_OUT), jnp.float32),
    )(acc, cnt, b1)


_E0_PAD = 163840
_E1_PAD = 16384
_S0_PAD = 10240
_S1_PAD = 1024


@functools.lru_cache(maxsize=None)
def _seg_sum(e_pad, dh, s_pad):
    return _make_seg_sum(e_pad, dh, s_pad)


def _pad_edges(src, dst, e, e_pad, dummy_dst, stride):
    nbt = e_pad // _BATCH
    roll = e_pad // (2 * _NS)
    srcp = jnp.concatenate([src, jnp.zeros((e_pad - e,), jnp.int32)])
    dstp = jnp.concatenate([dst, jnp.full((e_pad - e,), dummy_dst, jnp.int32)])
    srcq = jnp.stack([stride * srcp,
                      jnp.roll(stride * srcp + 1, roll)]).reshape(
        2, nbt, _BATCH)
    dstq = jnp.stack([dstp, jnp.roll(dstp, roll)]).reshape(2, nbt, _BATCH)
    return srcq, dstq


def kernel(features, src0, dst0, src1, dst1, W0, b0, W1, b1):
    src0q, dst0p = _pad_edges(src0, dst0, _E0, _E0_PAD, _N1, 2)
    src1q, dst1p = _pad_edges(src1, dst1, _E1, _E1_PAD, _N2, 4)
    zeros_h = jnp.zeros((_BATCH, _D_HID // 2), jnp.float32)
    zeros_o = jnp.zeros((_BATCH, _D_OUT // 2), jnp.float32)
    zeros_c = jnp.zeros((_BATCH, _CW), jnp.float32)
    ones_c = jnp.ones((_BATCH, _CW), jnp.float32)

    f = _matmul(features, W0)
    tbl0 = f.reshape(2 * _N0, _D_HID // 2)
    acc0, cnt0 = _seg_sum(_E0_PAD, _D_HID // 2, _S0_PAD)(
        tbl0, src0q, dst0p, zeros_h, zeros_c, ones_c)
    g = _mid(acc0, cnt0, b0.reshape(1, _D_HID),
             W1[:_D_HID], W1[_D_HID:])
    tbl1 = g.reshape(4 * _S0_PAD, _D_OUT // 2)
    acc1, cnt1 = _seg_sum(_E1_PAD, _D_OUT // 2, _S1_PAD)(
        tbl1, src1q, dst1p, zeros_o, zeros_c, ones_c)
    return _fin(acc1, cnt1, b1.reshape(1, _D_OUT))
